# baseline (speedup 1.0000x reference)
import functools

import jax
import jax.numpy as jnp
from jax import lax
from jax.experimental import pallas as pl
from jax.experimental.pallas import tpu as pltpu
from jax.experimental.pallas import tpu_sc as plsc

_N = 50000
_NP = 50048
_E = 800000
_EROWS_P = 6400
_ROWS_PER_W = 400
_G = 1
_NBUF = 6
_NITER = _ROWS_PER_W // _G
_WROWS = _NP // 16

_DH = 64
_DIN = 19
_DOUT = 32


def _make_seg():
  mesh = plsc.VectorSubcoreMesh(core_axis_name="c", subcore_axis_name="s")

  def body(xlo, xhi, src2d, dst2d, zrows, out_lo, out_hi,
           accum, src_b, dst_b, rows_b, gsem, ssem, isem):
    c = lax.axis_index("c")
    s = lax.axis_index("s")

    pltpu.sync_copy(zrows, accum.at[pl.ds(s * _WROWS, _WROWS)])
    plsc.subcore_barrier()

    def half(x_hbm, out_hbm):
      base = s * _ROWS_PER_W

      def idx_load(g, buf):
        row0 = base + g * _G
        pltpu.async_copy(src2d.at[pl.ds(row0, _G)], src_b.at[buf], isem)
        pltpu.async_copy(dst2d.at[pl.ds(row0, _G)], dst_b.at[buf], isem)

      def drain_scatters():
        for j in range(_G):
          pltpu.make_async_copy(zrows.at[pl.ds(0, 128)],
                                rows_b.at[0, j], ssem).wait()

      idx_load(0, 0)

      def step(g, carry):
        cur = lax.rem(g, _NBUF)
        nxt = lax.rem(g + 1, _NBUF)

        @pl.when(g + 1 < _NITER)
        def _():
          idx_load(g + 1, nxt)

        pltpu.make_async_copy(src2d.at[pl.ds(base, _G)],
                              src_b.at[cur], isem).wait()
        pltpu.make_async_copy(dst2d.at[pl.ds(base, _G)],
                              dst_b.at[cur], isem).wait()

        cps = [pltpu.async_copy(x_hbm.at[src_b.at[cur, j]],
                                rows_b.at[cur, j], gsem)
               for j in range(_G)]

        @pl.when(g >= _NBUF - 1)
        def _():
          drain_scatters()

        for cp in cps:
          cp.wait()
        for j in range(_G):
          pltpu.async_copy(rows_b.at[cur, j],
                           accum.at[dst_b.at[cur, j]], ssem, add=True)
        return carry

      lax.fori_loop(0, _NITER, step, 0)
      for _ in range(_NBUF - 1):
        drain_scatters()
      plsc.subcore_barrier()
      pltpu.sync_copy(accum.at[pl.ds(s * _WROWS, _WROWS)],
                      out_hbm.at[pl.ds(s * _WROWS, _WROWS)])

    @pl.when(c == 0)
    def _():
      half(xlo, out_lo)

    @pl.when(c == 1)
    def _():
      half(xhi, out_hi)

  out_type = [jax.ShapeDtypeStruct((_NP, 32), jnp.float32),
              jax.ShapeDtypeStruct((_NP, 32), jnp.float32)]
  scratch = [pltpu.VMEM_SHARED((_NP, 32), jnp.float32),
             pltpu.VMEM((_NBUF, _G, 128), jnp.int32),
             pltpu.VMEM((_NBUF, _G, 128), jnp.int32),
             pltpu.VMEM((_NBUF, _G, 128, 32), jnp.float32),
             pltpu.SemaphoreType.DMA,
             pltpu.SemaphoreType.DMA,
             pltpu.SemaphoreType.DMA]
  return pl.kernel(
      body, out_type=out_type, mesh=mesh, scratch_types=scratch,
      compiler_params=pltpu.CompilerParams(use_tc_tiling_on_sc=False))


_seg = _make_seg()

_CG = 8
_CROWS_PER_C = _EROWS_P // 2
_CROWS_PER_W = _CROWS_PER_C // 16
_CNITER = _CROWS_PER_W // _CG


def _make_cnt():
  mesh = plsc.VectorSubcoreMesh(core_axis_name="c", subcore_axis_name="s")

  def body(dst2d, zc16, ones16, cnt0, cnt1, accc, dst_b, ones_v, sem):
    c = lax.axis_index("c")
    s = lax.axis_index("s")
    pltpu.sync_copy(zc16, accc.at[pl.ds(s * _WROWS, _WROWS)])
    pltpu.sync_copy(ones16, ones_v)
    plsc.subcore_barrier()

    def half(out_hbm):
      def step(g, carry):
        row0 = c * _CROWS_PER_C + s * _CROWS_PER_W + g * _CG
        pltpu.sync_copy(dst2d.at[pl.ds(row0, _CG)], dst_b)
        for j in range(_CG):
          pltpu.sync_copy(ones_v, accc.at[dst_b.at[j]], add=True)
        return carry

      lax.fori_loop(0, _CNITER, step, 0)
      plsc.subcore_barrier()
      pltpu.sync_copy(accc.at[pl.ds(s * _WROWS, _WROWS)],
                      out_hbm.at[pl.ds(s * _WROWS, _WROWS)])

    @pl.when(c == 0)
    def _():
      half(cnt0)

    @pl.when(c == 1)
    def _():
      half(cnt1)

  out_type = [jax.ShapeDtypeStruct((_NP, 16), jnp.float32),
              jax.ShapeDtypeStruct((_NP, 16), jnp.float32)]
  scratch = [pltpu.VMEM_SHARED((_NP, 16), jnp.float32),
             pltpu.VMEM((_CG, 128), jnp.int32),
             pltpu.VMEM((128, 16), jnp.float32),
             pltpu.SemaphoreType.DMA]
  return pl.kernel(
      body, out_type=out_type, mesh=mesh, scratch_types=scratch,
      compiler_params=pltpu.CompilerParams(use_tc_tiling_on_sc=False))


_cnt = _make_cnt()


_R = 2000
_GRID = _N // _R


def _enc_body(nf, W1, b1, W2, b2, olo, ohi):
  h = jnp.dot(nf[...], W1[...], preferred_element_type=jnp.float32) + b1[...]
  h = jnp.maximum(h, 0.0)
  x = jnp.dot(h, W2[...], preferred_element_type=jnp.float32) + b2[...]
  olo[...] = x[:, :32]
  ohi[...] = x[:, 32:]


def _full(shape):
  return pl.BlockSpec(shape, lambda i: (0, 0))


def _encode(nf, W1, b1, W2, b2):
  return pl.pallas_call(
      _enc_body,
      grid=(_GRID,),
      in_specs=[pl.BlockSpec((_R, _DIN), lambda i: (i, 0)),
                _full((_DIN, _DH)), _full((1, _DH)),
                _full((_DH, _DH)), _full((1, _DH))],
      out_specs=[pl.BlockSpec((_R, 32), lambda i: (i, 0))] * 2,
      out_shape=[jax.ShapeDtypeStruct((_N, 32), jnp.float32)] * 2,
  )(nf, W1, b1, W2, b2)


def _comb_body(xlo, xhi, slo, shi, cnt0, cnt1, Wx, Wa, bg, gm, bt, olo, ohi):
  x = jnp.concatenate([xlo[...], xhi[...]], axis=1)
  ss = jnp.concatenate([slo[...], shi[...]], axis=1)
  c = jnp.sum(cnt0[...] + cnt1[...], axis=1, keepdims=True) * (1.0 / 16.0)
  inv = jnp.where(c > 0, 1.0 / jnp.maximum(c, 1.0), 0.0)
  agg = ss * inv
  h = (jnp.dot(x, Wx[...], preferred_element_type=jnp.float32)
       + jnp.dot(agg, Wa[...], preferred_element_type=jnp.float32)
       + bg[...])
  t = h + x
  mu = jnp.mean(t, axis=1, keepdims=True)
  var = jnp.mean((t - mu) * (t - mu), axis=1, keepdims=True)
  y = gm[...] * (t - mu) / jnp.sqrt(var + 1e-5) + bt[...]
  y = jnp.maximum(y, 0.0)
  olo[...] = y[:, :32]
  ohi[...] = y[:, 32:]


def _combine(xlo, xhi, slo, shi, cnt0, cnt1, Wx, Wa, bg, gm, bt):
  return pl.pallas_call(
      _comb_body,
      grid=(_GRID,),
      in_specs=[pl.BlockSpec((_R, 32), lambda i: (i, 0))] * 4
               + [pl.BlockSpec((_R, 16), lambda i: (i, 0))] * 2
               + [_full((_DH, _DH)), _full((_DH, _DH)),
                  _full((1, _DH)), _full((1, _DH)), _full((1, _DH))],
      out_specs=[pl.BlockSpec((_R, 32), lambda i: (i, 0))] * 2,
      out_shape=[jax.ShapeDtypeStruct((_N, 32), jnp.float32)] * 2,
  )(xlo, xhi, slo, shi, cnt0, cnt1, Wx, Wa, bg, gm, bt)


def _proj_body(xlo, xhi, Wo, bo, out):
  x = jnp.concatenate([xlo[...], xhi[...]], axis=1)
  out[...] = jnp.dot(x, Wo[...], preferred_element_type=jnp.float32) + bo[...]


def _project(xlo, xhi, Wo, bo):
  return pl.pallas_call(
      _proj_body,
      grid=(_GRID,),
      in_specs=[pl.BlockSpec((_R, 32), lambda i: (i, 0))] * 2
               + [_full((_DH, _DOUT)), _full((1, _DOUT))],
      out_specs=pl.BlockSpec((_R, _DOUT), lambda i: (i, 0)),
      out_shape=jax.ShapeDtypeStruct((_N, _DOUT), jnp.float32),
  )(xlo, xhi, Wo, bo)


def kernel(node_features, edge_indices, batch_size, W1, b1, W2, b2,
           Wg, bg, gamma, beta, Wo, bo):
  nf = node_features[0]
  ei = edge_indices[0].astype(jnp.int32)
  src, dst = ei[0], ei[1]
  pad = _EROWS_P * 128 - _E
  src2d = jnp.concatenate(
      [src, jnp.zeros((pad,), jnp.int32)]).reshape(_EROWS_P, 128)
  dst2d = jnp.concatenate(
      [dst, jnp.full((pad,), _N, jnp.int32)]).reshape(_EROWS_P, 128)
  zrows = jnp.zeros((_WROWS, 32), jnp.float32)
  zc16 = jnp.zeros((_WROWS, 16), jnp.float32)
  ones16 = jnp.ones((128, 16), jnp.float32)

  cnt0, cnt1 = _cnt(dst2d, zc16, ones16)
  xlo, xhi = _encode(nf, W1, b1.reshape(1, _DH), W2, b2.reshape(1, _DH))

  for l in range(3):
    slo, shi = _seg(xlo, xhi, src2d, dst2d, zrows)
    xlo, xhi = _combine(xlo, xhi, slo, shi, cnt0, cnt1,
                        Wg[l, :_DH], Wg[l, _DH:],
                        bg[l].reshape(1, _DH), gamma[l].reshape(1, _DH),
                        beta[l].reshape(1, _DH))

  out = _project(xlo, xhi, Wo, bo.reshape(1, _DOUT))
  return out[None]


# speedup vs baseline: 1.1465x; 1.1465x over previous
import functools

import jax
import jax.numpy as jnp
from jax import lax
from jax.experimental import pallas as pl
from jax.experimental.pallas import tpu as pltpu
from jax.experimental.pallas import tpu_sc as plsc

_N = 50000
_NP = 50048
_E = 800000
_EROWS_P = 6400
_ROWS_PER_W = 400
_G = 2
_NBUF = 3
_NITER = _ROWS_PER_W // _G
_WROWS = _NP // 16

_DH = 64
_DIN = 19
_DOUT = 32


def _make_seg():
  mesh = plsc.VectorSubcoreMesh(core_axis_name="c", subcore_axis_name="s")

  def body(xlo, xhi, src2d, dst2d, zrows, out_lo, out_hi,
           accum, src_b, dst_b, rows_b, gsem, ssem, isem):
    c = lax.axis_index("c")
    s = lax.axis_index("s")

    pltpu.sync_copy(zrows, accum.at[pl.ds(s * _WROWS, _WROWS)])
    plsc.subcore_barrier()

    def half(x_hbm, out_hbm):
      base = s * _ROWS_PER_W

      def idx_load(g, buf):
        row0 = base + g * _G
        pltpu.async_copy(src2d.at[pl.ds(row0, _G)], src_b.at[buf], isem)
        pltpu.async_copy(dst2d.at[pl.ds(row0, _G)], dst_b.at[buf], isem)

      def drain_scatters():
        for j in range(_G):
          pltpu.make_async_copy(zrows.at[pl.ds(0, 128)],
                                rows_b.at[0, j], ssem).wait()

      idx_load(0, 0)

      def step(g, carry):
        cur = lax.rem(g, _NBUF)
        nxt = lax.rem(g + 1, _NBUF)

        @pl.when(g + 1 < _NITER)
        def _():
          idx_load(g + 1, nxt)

        pltpu.make_async_copy(src2d.at[pl.ds(base, _G)],
                              src_b.at[cur], isem).wait()
        pltpu.make_async_copy(dst2d.at[pl.ds(base, _G)],
                              dst_b.at[cur], isem).wait()

        cps = [pltpu.async_copy(x_hbm.at[src_b.at[cur, j]],
                                rows_b.at[cur, j], gsem)
               for j in range(_G)]

        @pl.when(g >= _NBUF - 1)
        def _():
          drain_scatters()

        for cp in cps:
          cp.wait()
        for j in range(_G):
          pltpu.async_copy(rows_b.at[cur, j],
                           accum.at[dst_b.at[cur, j]], ssem, add=True)
        return carry

      lax.fori_loop(0, _NITER, step, 0)
      for _ in range(_NBUF - 1):
        drain_scatters()
      plsc.subcore_barrier()
      pltpu.sync_copy(accum.at[pl.ds(s * _WROWS, _WROWS)],
                      out_hbm.at[pl.ds(s * _WROWS, _WROWS)])

    @pl.when(c == 0)
    def _():
      half(xlo, out_lo)

    @pl.when(c == 1)
    def _():
      half(xhi, out_hi)

  out_type = [jax.ShapeDtypeStruct((_NP, 32), jnp.float32),
              jax.ShapeDtypeStruct((_NP, 32), jnp.float32)]
  scratch = [pltpu.VMEM_SHARED((_NP, 32), jnp.float32),
             pltpu.VMEM((_NBUF, _G, 128), jnp.int32),
             pltpu.VMEM((_NBUF, _G, 128), jnp.int32),
             pltpu.VMEM((_NBUF, _G, 128, 32), jnp.float32),
             pltpu.SemaphoreType.DMA,
             pltpu.SemaphoreType.DMA,
             pltpu.SemaphoreType.DMA]
  return pl.kernel(
      body, out_type=out_type, mesh=mesh, scratch_types=scratch,
      compiler_params=pltpu.CompilerParams(use_tc_tiling_on_sc=False))


_seg = _make_seg()

_CG = 8
_CROWS_PER_C = _EROWS_P // 2
_CROWS_PER_W = _CROWS_PER_C // 16
_CNITER = _CROWS_PER_W // _CG


def _make_cnt():
  mesh = plsc.VectorSubcoreMesh(core_axis_name="c", subcore_axis_name="s")

  def body(dst2d, zc16, ones16, cnt0, cnt1, accc, dst_b, ones_v, sem):
    c = lax.axis_index("c")
    s = lax.axis_index("s")
    pltpu.sync_copy(zc16, accc.at[pl.ds(s * _WROWS, _WROWS)])
    pltpu.sync_copy(ones16, ones_v)
    plsc.subcore_barrier()

    def half(out_hbm):
      def step(g, carry):
        row0 = c * _CROWS_PER_C + s * _CROWS_PER_W + g * _CG
        pltpu.sync_copy(dst2d.at[pl.ds(row0, _CG)], dst_b)
        for j in range(_CG):
          pltpu.sync_copy(ones_v, accc.at[dst_b.at[j]], add=True)
        return carry

      lax.fori_loop(0, _CNITER, step, 0)
      plsc.subcore_barrier()
      pltpu.sync_copy(accc.at[pl.ds(s * _WROWS, _WROWS)],
                      out_hbm.at[pl.ds(s * _WROWS, _WROWS)])

    @pl.when(c == 0)
    def _():
      half(cnt0)

    @pl.when(c == 1)
    def _():
      half(cnt1)

  out_type = [jax.ShapeDtypeStruct((_NP, 16), jnp.float32),
              jax.ShapeDtypeStruct((_NP, 16), jnp.float32)]
  scratch = [pltpu.VMEM_SHARED((_NP, 16), jnp.float32),
             pltpu.VMEM((_CG, 128), jnp.int32),
             pltpu.VMEM((128, 16), jnp.float32),
             pltpu.SemaphoreType.DMA]
  return pl.kernel(
      body, out_type=out_type, mesh=mesh, scratch_types=scratch,
      compiler_params=pltpu.CompilerParams(use_tc_tiling_on_sc=False))


_cnt = _make_cnt()


_R = 2000
_GRID = _N // _R


def _enc_body(nf, W1, b1, W2, b2, olo, ohi):
  h = jnp.dot(nf[...], W1[...], preferred_element_type=jnp.float32) + b1[...]
  h = jnp.maximum(h, 0.0)
  x = jnp.dot(h, W2[...], preferred_element_type=jnp.float32) + b2[...]
  olo[...] = x[:, :32]
  ohi[...] = x[:, 32:]


def _full(shape):
  return pl.BlockSpec(shape, lambda i: (0, 0))


def _encode(nf, W1, b1, W2, b2):
  return pl.pallas_call(
      _enc_body,
      grid=(_GRID,),
      in_specs=[pl.BlockSpec((_R, _DIN), lambda i: (i, 0)),
                _full((_DIN, _DH)), _full((1, _DH)),
                _full((_DH, _DH)), _full((1, _DH))],
      out_specs=[pl.BlockSpec((_R, 32), lambda i: (i, 0))] * 2,
      out_shape=[jax.ShapeDtypeStruct((_N, 32), jnp.float32)] * 2,
  )(nf, W1, b1, W2, b2)


def _comb_body(xlo, xhi, slo, shi, cnt0, cnt1, Wx, Wa, bg, gm, bt, olo, ohi):
  x = jnp.concatenate([xlo[...], xhi[...]], axis=1)
  ss = jnp.concatenate([slo[...], shi[...]], axis=1)
  c = jnp.sum(cnt0[...] + cnt1[...], axis=1, keepdims=True) * (1.0 / 16.0)
  inv = jnp.where(c > 0, 1.0 / jnp.maximum(c, 1.0), 0.0)
  agg = ss * inv
  h = (jnp.dot(x, Wx[...], preferred_element_type=jnp.float32)
       + jnp.dot(agg, Wa[...], preferred_element_type=jnp.float32)
       + bg[...])
  t = h + x
  mu = jnp.mean(t, axis=1, keepdims=True)
  var = jnp.mean((t - mu) * (t - mu), axis=1, keepdims=True)
  y = gm[...] * (t - mu) / jnp.sqrt(var + 1e-5) + bt[...]
  y = jnp.maximum(y, 0.0)
  olo[...] = y[:, :32]
  ohi[...] = y[:, 32:]


def _combine(xlo, xhi, slo, shi, cnt0, cnt1, Wx, Wa, bg, gm, bt):
  return pl.pallas_call(
      _comb_body,
      grid=(_GRID,),
      in_specs=[pl.BlockSpec((_R, 32), lambda i: (i, 0))] * 4
               + [pl.BlockSpec((_R, 16), lambda i: (i, 0))] * 2
               + [_full((_DH, _DH)), _full((_DH, _DH)),
                  _full((1, _DH)), _full((1, _DH)), _full((1, _DH))],
      out_specs=[pl.BlockSpec((_R, 32), lambda i: (i, 0))] * 2,
      out_shape=[jax.ShapeDtypeStruct((_N, 32), jnp.float32)] * 2,
  )(xlo, xhi, slo, shi, cnt0, cnt1, Wx, Wa, bg, gm, bt)


def _proj_body(xlo, xhi, Wo, bo, out):
  x = jnp.concatenate([xlo[...], xhi[...]], axis=1)
  out[...] = jnp.dot(x, Wo[...], preferred_element_type=jnp.float32) + bo[...]


def _project(xlo, xhi, Wo, bo):
  return pl.pallas_call(
      _proj_body,
      grid=(_GRID,),
      in_specs=[pl.BlockSpec((_R, 32), lambda i: (i, 0))] * 2
               + [_full((_DH, _DOUT)), _full((1, _DOUT))],
      out_specs=pl.BlockSpec((_R, _DOUT), lambda i: (i, 0)),
      out_shape=jax.ShapeDtypeStruct((_N, _DOUT), jnp.float32),
  )(xlo, xhi, Wo, bo)


def kernel(node_features, edge_indices, batch_size, W1, b1, W2, b2,
           Wg, bg, gamma, beta, Wo, bo):
  nf = node_features[0]
  ei = edge_indices[0].astype(jnp.int32)
  src, dst = ei[0], ei[1]
  pad = _EROWS_P * 128 - _E
  src2d = jnp.concatenate(
      [src, jnp.zeros((pad,), jnp.int32)]).reshape(_EROWS_P, 128)
  dst2d = jnp.concatenate(
      [dst, jnp.full((pad,), _N, jnp.int32)]).reshape(_EROWS_P, 128)
  zrows = jnp.zeros((_WROWS, 32), jnp.float32)
  zc16 = jnp.zeros((_WROWS, 16), jnp.float32)
  ones16 = jnp.ones((128, 16), jnp.float32)

  cnt0, cnt1 = _cnt(dst2d, zc16, ones16)
  xlo, xhi = _encode(nf, W1, b1.reshape(1, _DH), W2, b2.reshape(1, _DH))

  for l in range(3):
    slo, shi = _seg(xlo, xhi, src2d, dst2d, zrows)
    xlo, xhi = _combine(xlo, xhi, slo, shi, cnt0, cnt1,
                        Wg[l, :_DH], Wg[l, _DH:],
                        bg[l].reshape(1, _DH), gamma[l].reshape(1, _DH),
                        beta[l].reshape(1, _DH))

  out = _project(xlo, xhi, Wo, bo.reshape(1, _DOUT))
  return out[None]


# speedup vs baseline: 1.2614x; 1.1002x over previous
import functools

import jax
import jax.numpy as jnp
from jax import lax
from jax.experimental import pallas as pl
from jax.experimental.pallas import tpu as pltpu
from jax.experimental.pallas import tpu_sc as plsc

_N = 50000
_NP = 50048
_E = 800000
_EROWS_P = 6400
_ROWS_PER_W = 400
_G = 2
_NBUF = 3
_NITER = _ROWS_PER_W // _G
_WROWS = _NP // 16

_DH = 64
_DIN = 19
_DOUT = 32


def _make_seg():
  mesh = plsc.VectorSubcoreMesh(core_axis_name="c", subcore_axis_name="s")

  def body(xlo, xhi, src2d, dst2d, zrows, out_lo, out_hi,
           accum, src_b, dst_b, rows_b, gsem, ssem, isem):
    c = lax.axis_index("c")
    s = lax.axis_index("s")

    pltpu.sync_copy(zrows, accum.at[pl.ds(s * _WROWS, _WROWS)])
    plsc.subcore_barrier()

    def half(x_hbm, out_hbm):
      base = s * _ROWS_PER_W

      def idx_load(g, buf):
        row0 = base + g * _G
        pltpu.async_copy(src2d.at[pl.ds(row0, _G)], src_b.at[buf], isem)
        pltpu.async_copy(dst2d.at[pl.ds(row0, _G)], dst_b.at[buf], isem)

      def drain_scatters():
        for j in range(_G):
          pltpu.make_async_copy(zrows.at[pl.ds(0, 128)],
                                rows_b.at[0, j], ssem).wait()

      def wait_gathers():
        for j in range(_G):
          pltpu.make_async_copy(zrows.at[pl.ds(0, 128)],
                                rows_b.at[0, j], gsem).wait()

      def fire_scatters(buf):
        for j in range(_G):
          pltpu.async_copy(rows_b.at[buf, j],
                           accum.at[dst_b.at[buf, j]], ssem, add=True)

      idx_load(0, 0)

      def step(g, carry):
        cur = lax.rem(g, _NBUF)
        nxt = lax.rem(g + 1, _NBUF)
        prv = lax.rem(g + _NBUF - 1, _NBUF)

        @pl.when(g >= _NBUF - 1)
        def _():
          drain_scatters()

        @pl.when(g + 1 < _NITER)
        def _():
          idx_load(g + 1, nxt)

        pltpu.make_async_copy(src2d.at[pl.ds(base, _G)],
                              src_b.at[cur], isem).wait()
        pltpu.make_async_copy(dst2d.at[pl.ds(base, _G)],
                              dst_b.at[cur], isem).wait()

        for j in range(_G):
          pltpu.async_copy(x_hbm.at[src_b.at[cur, j]],
                           rows_b.at[cur, j], gsem)

        @pl.when(g > 0)
        def _():
          wait_gathers()
          fire_scatters(prv)
        return carry

      lax.fori_loop(0, _NITER, step, 0)
      last = (_NITER - 1) % _NBUF
      wait_gathers()
      fire_scatters(last)
      for _ in range(_NBUF - 1):
        drain_scatters()
      plsc.subcore_barrier()
      pltpu.sync_copy(accum.at[pl.ds(s * _WROWS, _WROWS)],
                      out_hbm.at[pl.ds(s * _WROWS, _WROWS)])

    @pl.when(c == 0)
    def _():
      half(xlo, out_lo)

    @pl.when(c == 1)
    def _():
      half(xhi, out_hi)

  out_type = [jax.ShapeDtypeStruct((_NP, 32), jnp.float32),
              jax.ShapeDtypeStruct((_NP, 32), jnp.float32)]
  scratch = [pltpu.VMEM_SHARED((_NP, 32), jnp.float32),
             pltpu.VMEM((_NBUF, _G, 128), jnp.int32),
             pltpu.VMEM((_NBUF, _G, 128), jnp.int32),
             pltpu.VMEM((_NBUF, _G, 128, 32), jnp.float32),
             pltpu.SemaphoreType.DMA,
             pltpu.SemaphoreType.DMA,
             pltpu.SemaphoreType.DMA]
  return pl.kernel(
      body, out_type=out_type, mesh=mesh, scratch_types=scratch,
      compiler_params=pltpu.CompilerParams(use_tc_tiling_on_sc=False))


_seg = _make_seg()

_CG = 8
_CROWS_PER_C = _EROWS_P // 2
_CROWS_PER_W = _CROWS_PER_C // 16
_CNITER = _CROWS_PER_W // _CG


def _make_cnt():
  mesh = plsc.VectorSubcoreMesh(core_axis_name="c", subcore_axis_name="s")

  def body(dst2d, zc16, ones16, cnt0, cnt1, accc, dst_b, ones_v, sem):
    c = lax.axis_index("c")
    s = lax.axis_index("s")
    pltpu.sync_copy(zc16, accc.at[pl.ds(s * _WROWS, _WROWS)])
    pltpu.sync_copy(ones16, ones_v)
    plsc.subcore_barrier()

    def half(out_hbm):
      def step(g, carry):
        row0 = c * _CROWS_PER_C + s * _CROWS_PER_W + g * _CG
        pltpu.sync_copy(dst2d.at[pl.ds(row0, _CG)], dst_b)
        for j in range(_CG):
          pltpu.sync_copy(ones_v, accc.at[dst_b.at[j]], add=True)
        return carry

      lax.fori_loop(0, _CNITER, step, 0)
      plsc.subcore_barrier()
      pltpu.sync_copy(accc.at[pl.ds(s * _WROWS, _WROWS)],
                      out_hbm.at[pl.ds(s * _WROWS, _WROWS)])

    @pl.when(c == 0)
    def _():
      half(cnt0)

    @pl.when(c == 1)
    def _():
      half(cnt1)

  out_type = [jax.ShapeDtypeStruct((_NP, 16), jnp.float32),
              jax.ShapeDtypeStruct((_NP, 16), jnp.float32)]
  scratch = [pltpu.VMEM_SHARED((_NP, 16), jnp.float32),
             pltpu.VMEM((_CG, 128), jnp.int32),
             pltpu.VMEM((128, 16), jnp.float32),
             pltpu.SemaphoreType.DMA]
  return pl.kernel(
      body, out_type=out_type, mesh=mesh, scratch_types=scratch,
      compiler_params=pltpu.CompilerParams(use_tc_tiling_on_sc=False))


_cnt = _make_cnt()


_R = 2000
_GRID = _N // _R


def _enc_body(nf, W1, b1, W2, b2, olo, ohi):
  h = jnp.dot(nf[...], W1[...], preferred_element_type=jnp.float32) + b1[...]
  h = jnp.maximum(h, 0.0)
  x = jnp.dot(h, W2[...], preferred_element_type=jnp.float32) + b2[...]
  olo[...] = x[:, :32]
  ohi[...] = x[:, 32:]


def _full(shape):
  return pl.BlockSpec(shape, lambda i: (0, 0))


def _encode(nf, W1, b1, W2, b2):
  return pl.pallas_call(
      _enc_body,
      grid=(_GRID,),
      in_specs=[pl.BlockSpec((_R, _DIN), lambda i: (i, 0)),
                _full((_DIN, _DH)), _full((1, _DH)),
                _full((_DH, _DH)), _full((1, _DH))],
      out_specs=[pl.BlockSpec((_R, 32), lambda i: (i, 0))] * 2,
      out_shape=[jax.ShapeDtypeStruct((_N, 32), jnp.float32)] * 2,
  )(nf, W1, b1, W2, b2)


def _comb_body(xlo, xhi, slo, shi, cnt0, cnt1, Wx, Wa, bg, gm, bt, olo, ohi):
  x = jnp.concatenate([xlo[...], xhi[...]], axis=1)
  ss = jnp.concatenate([slo[...], shi[...]], axis=1)
  c = jnp.sum(cnt0[...] + cnt1[...], axis=1, keepdims=True) * (1.0 / 16.0)
  inv = jnp.where(c > 0, 1.0 / jnp.maximum(c, 1.0), 0.0)
  agg = ss * inv
  h = (jnp.dot(x, Wx[...], preferred_element_type=jnp.float32)
       + jnp.dot(agg, Wa[...], preferred_element_type=jnp.float32)
       + bg[...])
  t = h + x
  mu = jnp.mean(t, axis=1, keepdims=True)
  var = jnp.mean((t - mu) * (t - mu), axis=1, keepdims=True)
  y = gm[...] * (t - mu) / jnp.sqrt(var + 1e-5) + bt[...]
  y = jnp.maximum(y, 0.0)
  olo[...] = y[:, :32]
  ohi[...] = y[:, 32:]


def _combine(xlo, xhi, slo, shi, cnt0, cnt1, Wx, Wa, bg, gm, bt):
  return pl.pallas_call(
      _comb_body,
      grid=(_GRID,),
      in_specs=[pl.BlockSpec((_R, 32), lambda i: (i, 0))] * 4
               + [pl.BlockSpec((_R, 16), lambda i: (i, 0))] * 2
               + [_full((_DH, _DH)), _full((_DH, _DH)),
                  _full((1, _DH)), _full((1, _DH)), _full((1, _DH))],
      out_specs=[pl.BlockSpec((_R, 32), lambda i: (i, 0))] * 2,
      out_shape=[jax.ShapeDtypeStruct((_N, 32), jnp.float32)] * 2,
  )(xlo, xhi, slo, shi, cnt0, cnt1, Wx, Wa, bg, gm, bt)


def _proj_body(xlo, xhi, Wo, bo, out):
  x = jnp.concatenate([xlo[...], xhi[...]], axis=1)
  out[...] = jnp.dot(x, Wo[...], preferred_element_type=jnp.float32) + bo[...]


def _project(xlo, xhi, Wo, bo):
  return pl.pallas_call(
      _proj_body,
      grid=(_GRID,),
      in_specs=[pl.BlockSpec((_R, 32), lambda i: (i, 0))] * 2
               + [_full((_DH, _DOUT)), _full((1, _DOUT))],
      out_specs=pl.BlockSpec((_R, _DOUT), lambda i: (i, 0)),
      out_shape=jax.ShapeDtypeStruct((_N, _DOUT), jnp.float32),
  )(xlo, xhi, Wo, bo)


def kernel(node_features, edge_indices, batch_size, W1, b1, W2, b2,
           Wg, bg, gamma, beta, Wo, bo):
  nf = node_features[0]
  ei = edge_indices[0].astype(jnp.int32)
  src, dst = ei[0], ei[1]
  pad = _EROWS_P * 128 - _E
  src2d = jnp.concatenate(
      [src, jnp.zeros((pad,), jnp.int32)]).reshape(_EROWS_P, 128)
  dst2d = jnp.concatenate(
      [dst, jnp.full((pad,), _N, jnp.int32)]).reshape(_EROWS_P, 128)
  zrows = jnp.zeros((_WROWS, 32), jnp.float32)
  zc16 = jnp.zeros((_WROWS, 16), jnp.float32)
  ones16 = jnp.ones((128, 16), jnp.float32)

  cnt0, cnt1 = _cnt(dst2d, zc16, ones16)
  xlo, xhi = _encode(nf, W1, b1.reshape(1, _DH), W2, b2.reshape(1, _DH))

  for l in range(3):
    slo, shi = _seg(xlo, xhi, src2d, dst2d, zrows)
    xlo, xhi = _combine(xlo, xhi, slo, shi, cnt0, cnt1,
                        Wg[l, :_DH], Wg[l, _DH:],
                        bg[l].reshape(1, _DH), gamma[l].reshape(1, _DH),
                        beta[l].reshape(1, _DH))

  out = _project(xlo, xhi, Wo, bo.reshape(1, _DOUT))
  return out[None]


# speedup vs baseline: 2.1392x; 1.6960x over previous
import functools

import jax
import jax.numpy as jnp
from jax import lax
from jax.experimental import pallas as pl
from jax.experimental.pallas import tpu as pltpu
from jax.experimental.pallas import tpu_sc as plsc

_N = 50000
_NP = 50048
_E = 800000
_EROWS = 6250
_G = 2
_NBUF = 3
_WROWS = _NP // 16

_DH = 64
_DIN = 19
_DOUT = 32


def _make_seg():
  mesh = plsc.VectorSubcoreMesh(core_axis_name="c", subcore_axis_name="s")

  def body(xlo, xhi, src2d, dst2d, zrows, out_lo, out_hi,
           accum, src_b, dst_b, rows_b, gsem, ssem, isem):
    c = lax.axis_index("c")
    s = lax.axis_index("s")

    pltpu.sync_copy(zrows, accum.at[pl.ds(s * _WROWS, _WROWS)])
    plsc.subcore_barrier()

    def half(x_hbm, out_hbm):
      base = s * 390
      niter = jnp.where(s == 15, 200, 195)

      def idx_load(g, buf):
        row0 = base + g * _G
        pltpu.async_copy(src2d.at[pl.ds(row0, _G)], src_b.at[buf], isem)
        pltpu.async_copy(dst2d.at[pl.ds(row0, _G)], dst_b.at[buf], isem)

      def drain_scatters():
        for j in range(_G):
          pltpu.make_async_copy(zrows.at[pl.ds(0, 128)],
                                rows_b.at[0, j], ssem).wait()

      def wait_gathers():
        for j in range(_G):
          pltpu.make_async_copy(zrows.at[pl.ds(0, 128)],
                                rows_b.at[0, j], gsem).wait()

      def fire_scatters(buf):
        for j in range(_G):
          pltpu.async_copy(rows_b.at[buf, j],
                           accum.at[dst_b.at[buf, j]], ssem, add=True)

      idx_load(0, 0)

      def step(g, carry):
        cur = lax.rem(g, _NBUF)
        nxt = lax.rem(g + 1, _NBUF)
        prv = lax.rem(g + _NBUF - 1, _NBUF)

        @pl.when(g >= _NBUF - 1)
        def _():
          drain_scatters()

        @pl.when(g + 1 < niter)
        def _():
          idx_load(g + 1, nxt)

        pltpu.make_async_copy(src2d.at[pl.ds(base, _G)],
                              src_b.at[cur], isem).wait()
        pltpu.make_async_copy(dst2d.at[pl.ds(base, _G)],
                              dst_b.at[cur], isem).wait()

        for j in range(_G):
          pltpu.async_copy(x_hbm.at[src_b.at[cur, j]],
                           rows_b.at[cur, j], gsem)

        @pl.when(g > 0)
        def _():
          wait_gathers()
          fire_scatters(prv)
        return carry

      lax.fori_loop(0, niter, step, 0)
      wait_gathers()
      fire_scatters(lax.rem(niter - 1, _NBUF))
      for _ in range(_NBUF - 1):
        drain_scatters()
      plsc.subcore_barrier()
      pltpu.sync_copy(accum.at[pl.ds(s * _WROWS, _WROWS)],
                      out_hbm.at[pl.ds(s * _WROWS, _WROWS)])

    @pl.when(c == 0)
    def _():
      half(xlo, out_lo)

    @pl.when(c == 1)
    def _():
      half(xhi, out_hi)

  out_type = [jax.ShapeDtypeStruct((_NP, 32), jnp.float32),
              jax.ShapeDtypeStruct((_NP, 32), jnp.float32)]
  scratch = [pltpu.VMEM_SHARED((_NP, 32), jnp.float32),
             pltpu.VMEM((_NBUF, _G, 128), jnp.int32),
             pltpu.VMEM((_NBUF, _G, 128), jnp.int32),
             pltpu.VMEM((_NBUF, _G, 128, 32), jnp.float32),
             pltpu.SemaphoreType.DMA,
             pltpu.SemaphoreType.DMA,
             pltpu.SemaphoreType.DMA]
  return pl.kernel(
      body, out_type=out_type, mesh=mesh, scratch_types=scratch,
      compiler_params=pltpu.CompilerParams(use_tc_tiling_on_sc=False))


_seg = _make_seg()

_CG = 2


def _make_cnt():
  mesh = plsc.VectorSubcoreMesh(core_axis_name="c", subcore_axis_name="s")

  def body(dst2d, zc16, ones16, cnt0, cnt1, accc, dst_b, ones_v, sem):
    c = lax.axis_index("c")
    s = lax.axis_index("s")
    pltpu.sync_copy(zc16, accc.at[pl.ds(s * _WROWS, _WROWS)])
    pltpu.sync_copy(ones16, ones_v)
    plsc.subcore_barrier()

    w = c * 16 + s
    base = w * 200
    niter = jnp.where(w == 31, 25, 100)

    def step(g, carry):
      row0 = base + g * _CG
      pltpu.sync_copy(dst2d.at[pl.ds(row0, _CG)], dst_b)
      for j in range(_CG):
        pltpu.sync_copy(ones_v, accc.at[dst_b.at[j]], add=True)
      return carry

    lax.fori_loop(0, niter, step, 0)
    plsc.subcore_barrier()

    @pl.when(c == 0)
    def _():
      pltpu.sync_copy(accc.at[pl.ds(s * _WROWS, _WROWS)],
                      cnt0.at[pl.ds(s * _WROWS, _WROWS)])

    @pl.when(c == 1)
    def _():
      pltpu.sync_copy(accc.at[pl.ds(s * _WROWS, _WROWS)],
                      cnt1.at[pl.ds(s * _WROWS, _WROWS)])

  out_type = [jax.ShapeDtypeStruct((_NP, 16), jnp.float32),
              jax.ShapeDtypeStruct((_NP, 16), jnp.float32)]
  scratch = [pltpu.VMEM_SHARED((_NP, 16), jnp.float32),
             pltpu.VMEM((_CG, 128), jnp.int32),
             pltpu.VMEM((128, 16), jnp.float32),
             pltpu.SemaphoreType.DMA]
  return pl.kernel(
      body, out_type=out_type, mesh=mesh, scratch_types=scratch,
      compiler_params=pltpu.CompilerParams(use_tc_tiling_on_sc=False))


_cnt = _make_cnt()


_R = 2000
_GRID = _N // _R


def _enc_body(nf, W1, b1, W2, b2, olo, ohi):
  h = jnp.dot(nf[...], W1[...], preferred_element_type=jnp.float32) + b1[...]
  h = jnp.maximum(h, 0.0)
  x = jnp.dot(h, W2[...], preferred_element_type=jnp.float32) + b2[...]
  olo[...] = x[:, :32]
  ohi[...] = x[:, 32:]


def _full(shape):
  return pl.BlockSpec(shape, lambda i: (0, 0))


def _encode(nf, W1, b1, W2, b2):
  return pl.pallas_call(
      _enc_body,
      grid=(_GRID,),
      in_specs=[pl.BlockSpec((_R, _DIN), lambda i: (i, 0)),
                _full((_DIN, _DH)), _full((1, _DH)),
                _full((_DH, _DH)), _full((1, _DH))],
      out_specs=[pl.BlockSpec((_R, 32), lambda i: (i, 0))] * 2,
      out_shape=[jax.ShapeDtypeStruct((_N, 32), jnp.float32)] * 2,
  )(nf, W1, b1, W2, b2)


def _comb_body(xlo, xhi, slo, shi, cnt0, cnt1, Wx, Wa, bg, gm, bt, olo, ohi):
  x = jnp.concatenate([xlo[...], xhi[...]], axis=1)
  ss = jnp.concatenate([slo[...], shi[...]], axis=1)
  c = jnp.sum(cnt0[...] + cnt1[...], axis=1, keepdims=True) * (1.0 / 16.0)
  inv = jnp.where(c > 0, 1.0 / jnp.maximum(c, 1.0), 0.0)
  agg = ss * inv
  h = (jnp.dot(x, Wx[...], preferred_element_type=jnp.float32)
       + jnp.dot(agg, Wa[...], preferred_element_type=jnp.float32)
       + bg[...])
  t = h + x
  mu = jnp.mean(t, axis=1, keepdims=True)
  var = jnp.mean((t - mu) * (t - mu), axis=1, keepdims=True)
  y = gm[...] * (t - mu) / jnp.sqrt(var + 1e-5) + bt[...]
  y = jnp.maximum(y, 0.0)
  olo[...] = y[:, :32]
  ohi[...] = y[:, 32:]


def _combine(xlo, xhi, slo, shi, cnt0, cnt1, Wx, Wa, bg, gm, bt):
  return pl.pallas_call(
      _comb_body,
      grid=(_GRID,),
      in_specs=[pl.BlockSpec((_R, 32), lambda i: (i, 0))] * 4
               + [pl.BlockSpec((_R, 16), lambda i: (i, 0))] * 2
               + [_full((_DH, _DH)), _full((_DH, _DH)),
                  _full((1, _DH)), _full((1, _DH)), _full((1, _DH))],
      out_specs=[pl.BlockSpec((_R, 32), lambda i: (i, 0))] * 2,
      out_shape=[jax.ShapeDtypeStruct((_N, 32), jnp.float32)] * 2,
  )(xlo, xhi, slo, shi, cnt0, cnt1, Wx, Wa, bg, gm, bt)


def _comb_proj_body(xlo, xhi, slo, shi, cnt0, cnt1, Wx, Wa, bg, gm, bt,
                    Wo, bo, out):
  x = jnp.concatenate([xlo[...], xhi[...]], axis=1)
  ss = jnp.concatenate([slo[...], shi[...]], axis=1)
  c = jnp.sum(cnt0[...] + cnt1[...], axis=1, keepdims=True) * (1.0 / 16.0)
  inv = jnp.where(c > 0, 1.0 / jnp.maximum(c, 1.0), 0.0)
  agg = ss * inv
  h = (jnp.dot(x, Wx[...], preferred_element_type=jnp.float32)
       + jnp.dot(agg, Wa[...], preferred_element_type=jnp.float32)
       + bg[...])
  t = h + x
  mu = jnp.mean(t, axis=1, keepdims=True)
  var = jnp.mean((t - mu) * (t - mu), axis=1, keepdims=True)
  y = gm[...] * (t - mu) / jnp.sqrt(var + 1e-5) + bt[...]
  y = jnp.maximum(y, 0.0)
  out[...] = jnp.dot(y, Wo[...], preferred_element_type=jnp.float32) + bo[...]


def _combine_project(xlo, xhi, slo, shi, cnt0, cnt1, Wx, Wa, bg, gm, bt,
                     Wo, bo):
  return pl.pallas_call(
      _comb_proj_body,
      grid=(_GRID,),
      in_specs=[pl.BlockSpec((_R, 32), lambda i: (i, 0))] * 4
               + [pl.BlockSpec((_R, 16), lambda i: (i, 0))] * 2
               + [_full((_DH, _DH)), _full((_DH, _DH)),
                  _full((1, _DH)), _full((1, _DH)), _full((1, _DH)),
                  _full((_DH, _DOUT)), _full((1, _DOUT))],
      out_specs=pl.BlockSpec((_R, _DOUT), lambda i: (i, 0)),
      out_shape=jax.ShapeDtypeStruct((_N, _DOUT), jnp.float32),
  )(xlo, xhi, slo, shi, cnt0, cnt1, Wx, Wa, bg, gm, bt, Wo, bo)


def kernel(node_features, edge_indices, batch_size, W1, b1, W2, b2,
           Wg, bg, gamma, beta, Wo, bo):
  nf = node_features[0]
  ei = edge_indices[0].astype(jnp.int32)
  src2d = ei[0].reshape(_EROWS, 128)
  dst2d = ei[1].reshape(_EROWS, 128)
  zrows = jnp.zeros((_WROWS, 32), jnp.float32)
  zc16 = jnp.zeros((_WROWS, 16), jnp.float32)
  ones16 = jnp.ones((128, 16), jnp.float32)

  cnt0, cnt1 = _cnt(dst2d, zc16, ones16)
  xlo, xhi = _encode(nf, W1, b1.reshape(1, _DH), W2, b2.reshape(1, _DH))

  for l in range(2):
    slo, shi = _seg(xlo, xhi, src2d, dst2d, zrows)
    xlo, xhi = _combine(xlo, xhi, slo, shi, cnt0, cnt1,
                        Wg[l, :_DH], Wg[l, _DH:],
                        bg[l].reshape(1, _DH), gamma[l].reshape(1, _DH),
                        beta[l].reshape(1, _DH))

  slo, shi = _seg(xlo, xhi, src2d, dst2d, zrows)
  out = _combine_project(xlo, xhi, slo, shi, cnt0, cnt1,
                         Wg[2, :_DH], Wg[2, _DH:],
                         bg[2].reshape(1, _DH), gamma[2].reshape(1, _DH),
                         beta[2].reshape(1, _DH), Wo, bo.reshape(1, _DOUT))
  return out[None]
